# Initial kernel scaffold; baseline (speedup 1.0000x reference)
import jax
import jax.numpy as jnp
from jax.experimental import pallas as pl


def kernel(feat, efeat, edge_index, W_vsk, b_vsk, W_vrk, b_vrk, W_ek, b_ek, W1, b1, W2, b2):
    raise NotImplementedError("write your pallas kernel here")


# speedup vs baseline: 2.4496x; 2.4496x over previous
import functools

import jax
import jax.numpy as jnp
from jax import lax
from jax.experimental import pallas as pl
from jax.experimental.pallas import tpu as pltpu
from jax.experimental.pallas import tpu_sc as plsc

_NC = 2
_NS = 16
_L = 16
_NW = _NC * _NS


def _node_proj_body(feat_ref, w_ref, b_ref, t_ref):
    t_ref[...] = (
        jnp.dot(feat_ref[...], w_ref[...], preferred_element_type=jnp.float32)
        + b_ref[...]
    )


def _edge_mlp_body(vk2_ref, eflo_ref, efhi_ref, wek_ref, bek_ref, w1_ref, b1_ref,
                   w2_ref, b2_ref, out_ref):
    o = out_ref.shape[2]
    efc = jnp.concatenate([eflo_ref[...], efhi_ref[...]], axis=1)
    ekc = jnp.dot(efc, wek_ref[...], preferred_element_type=jnp.float32)
    a = jnp.maximum(vk2_ref[...] + ekc + bek_ref[...], 0.0)
    a = jnp.maximum(
        jnp.dot(a, w1_ref[...], preferred_element_type=jnp.float32) + b1_ref[...], 0.0
    )
    a = jnp.maximum(
        jnp.dot(a, w2_ref[...], preferred_element_type=jnp.float32) + b2_ref[...], 0.0
    )
    out_ref[0] = a[:, :o]
    out_ref[1] = a[:, o:]


def _make_gather_kernel(n_nodes, n_edges, h, epw, ch):
    nch = epw // ch
    h2 = 2 * h
    half = n_edges // 2
    rows_per_sub = (n_nodes // _NS) // 8 * 8
    mesh = plsc.VectorSubcoreMesh(
        core_axis_name="c", subcore_axis_name="s", num_cores=_NC, num_subcores=_NS
    )

    @functools.partial(
        pl.kernel,
        out_type=jax.ShapeDtypeStruct((half, h2), jnp.float32),
        mesh=mesh,
        scratch_types=[
            pltpu.VMEM_SHARED((n_nodes, h2), jnp.float32),
            pltpu.VMEM((ch,), jnp.int32),
            pltpu.VMEM((ch,), jnp.int32),
            pltpu.VMEM((ch,), jnp.int32),
            pltpu.VMEM((ch,), jnp.int32),
            pltpu.VMEM((ch, h2), jnp.float32),
            pltpu.VMEM((ch, h2), jnp.float32),
            pltpu.VMEM((ch, h2), jnp.float32),
            pltpu.VMEM((ch, h2), jnp.float32),
            pltpu.SemaphoreType.DMA,
        ],
    )
    def gather_add(t_hbm, src_hbm, dst_hbm, out_hbm,
                   t_sh, sl_v, dl_v, sh_v, dh_v, rsl_v, rdl_v, rsh_v, rdh_v,
                   sem):
        cid = lax.axis_index("c")
        sid = lax.axis_index("s")
        wid = sid * _NC + cid
        base = wid * epw

        for s in range(_NS):
            sz = rows_per_sub if s < _NS - 1 else n_nodes - rows_per_sub * (_NS - 1)

            @pl.when(sid == s)
            def _stage(s=s, sz=sz):
                pltpu.sync_copy(
                    t_hbm.at[pl.ds(s * rows_per_sub, sz)],
                    t_sh.at[pl.ds(s * rows_per_sub, sz)],
                )

        plsc.subcore_barrier()

        def chunk(i, carry):
            off = base + i * ch
            pltpu.sync_copy(src_hbm.at[pl.ds(off, ch)], sl_v)
            pltpu.sync_copy(dst_hbm.at[pl.ds(off, ch)], dl_v)
            pltpu.sync_copy(src_hbm.at[pl.ds(half + off, ch)], sh_v)
            pltpu.sync_copy(dst_hbm.at[pl.ds(half + off, ch)], dh_v)
            c0 = pltpu.async_copy(t_sh.at[sl_v], rsl_v, sem)
            c1 = pltpu.async_copy(t_sh.at[dl_v], rdl_v, sem)
            c2 = pltpu.async_copy(t_sh.at[sh_v], rsh_v, sem)
            c3 = pltpu.async_copy(t_sh.at[dh_v], rdh_v, sem)
            c0.wait()
            c1.wait()
            c2.wait()
            c3.wait()

            @plsc.parallel_loop(0, ch, unroll=4)
            def add_rows(r):
                for c in range(h // _L):
                    lo = pl.ds(c * _L, _L)
                    hi = pl.ds(h + c * _L, _L)
                    rsl_v[r, lo] = rsl_v[r, lo] + rdl_v[r, hi]
                    rsl_v[r, hi] = rsh_v[r, lo] + rdh_v[r, hi]

            pltpu.sync_copy(rsl_v, out_hbm.at[pl.ds(pl.multiple_of(off, 8), ch)])
            return carry

        lax.fori_loop(0, nch, chunk, 0)

    return gather_add


def _blkdiag(w):
    r, c = w.shape
    z = jnp.zeros((2 * r, 2 * c), w.dtype)
    return z.at[:r, :c].set(w).at[r:, c:].set(w)


def kernel(feat, efeat, edge_index, W_vsk, b_vsk, W_vrk, b_vrk, W_ek, b_ek, W1, b1,
           W2, b2):
    n, f_in = feat.shape
    e = efeat.shape[0]
    h = W_vsk.shape[0]
    o = W2.shape[0]
    half = e // 2

    w_cat = jnp.concatenate([W_vsk.T, W_vrk.T], axis=1)
    b_cat = jnp.concatenate([b_vsk, b_vrk])[None, :]
    t_tab = pl.pallas_call(
        _node_proj_body,
        out_shape=jax.ShapeDtypeStruct((n, 2 * h), jnp.float32),
    )(feat, w_cat, b_cat)

    epw = half // _NW
    ch = 40
    gather_add = _make_gather_kernel(n, e, h, epw, ch)
    vk2 = gather_add(t_tab, edge_index[0], edge_index[1])

    beh = 1600
    nblk = half // beh
    wek_d = _blkdiag(W_ek.T)
    bek_d = jnp.concatenate([b_ek, b_ek])[None, :]
    w1_d = _blkdiag(W1.T)
    b1_d = jnp.concatenate([b1, b1])[None, :]
    w2_d = _blkdiag(W2.T)
    b2_d = jnp.concatenate([b2, b2])[None, :]
    out2 = pl.pallas_call(
        _edge_mlp_body,
        grid=(nblk,),
        in_specs=[
            pl.BlockSpec((beh, 2 * h), lambda i: (i, 0)),
            pl.BlockSpec((beh, f_in), lambda i: (i, 0)),
            pl.BlockSpec((beh, f_in), lambda i, _n=nblk: (i + _n, 0)),
            pl.BlockSpec((2 * f_in, 2 * h), lambda i: (0, 0)),
            pl.BlockSpec((1, 2 * h), lambda i: (0, 0)),
            pl.BlockSpec((2 * h, 2 * h), lambda i: (0, 0)),
            pl.BlockSpec((1, 2 * h), lambda i: (0, 0)),
            pl.BlockSpec((2 * h, 2 * o), lambda i: (0, 0)),
            pl.BlockSpec((1, 2 * o), lambda i: (0, 0)),
        ],
        out_specs=pl.BlockSpec((2, beh, o), lambda i: (0, i, 0)),
        out_shape=jax.ShapeDtypeStruct((2, half, o), jnp.float32),
        compiler_params=pltpu.CompilerParams(
            dimension_semantics=("arbitrary",),
        ),
    )(vk2, efeat, efeat, wek_d, bek_d, w1_d, b1_d, w2_d, b2_d)
    return out2.reshape(e, o)


# speedup vs baseline: 2.7438x; 1.1201x over previous
import functools

import jax
import jax.numpy as jnp
from jax import lax
from jax.experimental import pallas as pl
from jax.experimental.pallas import tpu as pltpu
from jax.experimental.pallas import tpu_sc as plsc

_NC = 2
_NS = 16
_L = 16
_NW = _NC * _NS


def _node_proj_body(feat_ref, wsk_ref, bsk_ref, wrk_ref, brk_ref, vsk_ref, vrk_ref):
    f = feat_ref[...]
    vsk_ref[...] = (
        jnp.dot(f, wsk_ref[...], preferred_element_type=jnp.float32) + bsk_ref[...]
    )
    vrk_ref[...] = (
        jnp.dot(f, wrk_ref[...], preferred_element_type=jnp.float32) + brk_ref[...]
    )


def _edge_mlp_body(vk2_ref, eflo_ref, efhi_ref, wek_ref, bek_ref, w1_ref, b1_ref,
                   w2_ref, b2_ref, out_ref):
    o = out_ref.shape[2]
    efc = jnp.concatenate([eflo_ref[...], efhi_ref[...]], axis=1)
    ekc = jnp.dot(efc, wek_ref[...], preferred_element_type=jnp.float32)
    a = jnp.maximum(vk2_ref[...] + ekc + bek_ref[...], 0.0)
    a = jnp.maximum(
        jnp.dot(a, w1_ref[...], preferred_element_type=jnp.float32) + b1_ref[...], 0.0
    )
    a = jnp.maximum(
        jnp.dot(a, w2_ref[...], preferred_element_type=jnp.float32) + b2_ref[...], 0.0
    )
    out_ref[0] = a[:, :o]
    out_ref[1] = a[:, o:]


def _make_gather_kernel(n_nodes, n_edges, h, epw, ch):
    nch = epw // ch
    ch2 = 2 * ch
    h2 = 2 * h
    rps = (n_nodes // _NS) // 8 * 8
    mesh = plsc.VectorSubcoreMesh(
        core_axis_name="c", subcore_axis_name="s", num_cores=_NC, num_subcores=_NS
    )

    @functools.partial(
        pl.kernel,
        out_type=jax.ShapeDtypeStruct((n_edges // 2, h2), jnp.float32),
        mesh=mesh,
        scratch_types=[
            pltpu.VMEM_SHARED((n_nodes, h), jnp.float32),
            pltpu.VMEM_SHARED((n_nodes, h), jnp.float32),
            pltpu.VMEM((2 * epw,), jnp.int32),
            pltpu.VMEM((2 * epw,), jnp.int32),
            pltpu.VMEM((ch2, h), jnp.float32),
            pltpu.VMEM((ch2, h), jnp.float32),
            pltpu.VMEM((ch2, h), jnp.float32),
            pltpu.VMEM((ch2, h), jnp.float32),
            pltpu.VMEM((ch, h2), jnp.float32),
            pltpu.VMEM((ch, h2), jnp.float32),
            pltpu.SemaphoreType.DMA,
            pltpu.SemaphoreType.DMA,
            pltpu.SemaphoreType.DMA,
        ],
        compiler_params=pltpu.CompilerParams(use_tc_tiling_on_sc=False),
    )
    def gather_add(vsk_hbm, vrk_hbm, srcp_hbm, dstp_hbm, out_hbm,
                   vsk_sh, vrk_sh, sidx, didx, rs0, rd0, rs1, rd1, ov0, ov1,
                   semg0, semg1, semo):
        cid = lax.axis_index("c")
        sid = lax.axis_index("s")
        wid = sid * _NC + cid
        base = wid * epw

        for s in range(_NS):
            sz = rps if s < _NS - 1 else n_nodes - rps * (_NS - 1)

            @pl.when(sid == s)
            def _stage(s=s, sz=sz):
                pltpu.sync_copy(vsk_hbm.at[pl.ds(s * rps, sz)],
                                vsk_sh.at[pl.ds(s * rps, sz)])
                pltpu.sync_copy(vrk_hbm.at[pl.ds(s * rps, sz)],
                                vrk_sh.at[pl.ds(s * rps, sz)])

        plsc.subcore_barrier()

        pltpu.sync_copy(srcp_hbm.at[pl.ds(2 * base, 2 * epw)], sidx)
        pltpu.sync_copy(dstp_hbm.at[pl.ds(2 * base, 2 * epw)], didx)

        def issue(j, rs, rd, semg):
            io = pl.multiple_of(j * ch2, 8)
            cs = pltpu.async_copy(vsk_sh.at[sidx.at[pl.ds(io, ch2)]], rs, semg)
            cd = pltpu.async_copy(vrk_sh.at[didx.at[pl.ds(io, ch2)]], rd, semg)
            return cs, cd

        def combine(rs, rd, ov):
            @plsc.parallel_loop(0, ch, unroll=4)
            def add_rows(r):
                for c in range(h // _L):
                    sl = pl.ds(c * _L, _L)
                    sr = pl.ds(h + c * _L, _L)
                    ov[r, sl] = rs[2 * r, sl] + rd[2 * r, sl]
                    ov[r, sr] = rs[2 * r + 1, sl] + rd[2 * r + 1, sl]

        def flush(j, ov):
            oo = pl.multiple_of(base + j * ch, 8)
            return pltpu.async_copy(ov, out_hbm.at[pl.ds(oo, ch)], semo)

        def pair_body(i, carry):
            j0 = 2 * i
            j1 = 2 * i + 1
            g0 = issue(j0, rs0, rd0, semg0)
            g1 = issue(j1, rs1, rd1, semg1)
            g0[0].wait()
            g0[1].wait()
            combine(rs0, rd0, ov0)
            o0 = flush(j0, ov0)
            g1[0].wait()
            g1[1].wait()
            combine(rs1, rd1, ov1)
            o1 = flush(j1, ov1)
            o0.wait()
            o1.wait()
            return carry

        lax.fori_loop(0, nch // 2, pair_body, 0)

        if nch % 2:
            j = nch - 1
            g = issue(j, rs0, rd0, semg0)
            g[0].wait()
            g[1].wait()
            combine(rs0, rd0, ov0)
            flush(j, ov0).wait()

    return gather_add


def _blkdiag(w):
    r, c = w.shape
    z = jnp.zeros((2 * r, 2 * c), w.dtype)
    return z.at[:r, :c].set(w).at[r:, c:].set(w)


def kernel(feat, efeat, edge_index, W_vsk, b_vsk, W_vrk, b_vrk, W_ek, b_ek, W1, b1,
           W2, b2):
    n, f_in = feat.shape
    e = efeat.shape[0]
    h = W_vsk.shape[0]
    o = W2.shape[0]
    half = e // 2

    vsk, vrk = pl.pallas_call(
        _node_proj_body,
        out_shape=[
            jax.ShapeDtypeStruct((n, h), jnp.float32),
            jax.ShapeDtypeStruct((n, h), jnp.float32),
        ],
    )(feat, W_vsk.T, b_vsk[None, :], W_vrk.T, b_vrk[None, :])

    src2 = edge_index[0].reshape(2, half).T.reshape(-1)
    dst2 = edge_index[1].reshape(2, half).T.reshape(-1)
    epw = half // _NW
    ch = 40
    gather_add = _make_gather_kernel(n, e, h, epw, ch)
    vk2 = gather_add(vsk, vrk, src2, dst2)

    beh = 1600
    nblk = half // beh
    wek_d = _blkdiag(W_ek.T)
    bek_d = jnp.concatenate([b_ek, b_ek])[None, :]
    w1_d = _blkdiag(W1.T)
    b1_d = jnp.concatenate([b1, b1])[None, :]
    w2_d = _blkdiag(W2.T)
    b2_d = jnp.concatenate([b2, b2])[None, :]
    out2 = pl.pallas_call(
        _edge_mlp_body,
        grid=(nblk,),
        in_specs=[
            pl.BlockSpec((beh, 2 * h), lambda i: (i, 0)),
            pl.BlockSpec((beh, f_in), lambda i: (i, 0)),
            pl.BlockSpec((beh, f_in), lambda i, _n=nblk: (i + _n, 0)),
            pl.BlockSpec((2 * f_in, 2 * h), lambda i: (0, 0)),
            pl.BlockSpec((1, 2 * h), lambda i: (0, 0)),
            pl.BlockSpec((2 * h, 2 * h), lambda i: (0, 0)),
            pl.BlockSpec((1, 2 * h), lambda i: (0, 0)),
            pl.BlockSpec((2 * h, 2 * o), lambda i: (0, 0)),
            pl.BlockSpec((1, 2 * o), lambda i: (0, 0)),
        ],
        out_specs=pl.BlockSpec((2, beh, o), lambda i: (0, i, 0)),
        out_shape=jax.ShapeDtypeStruct((2, half, o), jnp.float32),
        compiler_params=pltpu.CompilerParams(
            dimension_semantics=("arbitrary",),
        ),
    )(vk2, efeat, efeat, wek_d, bek_d, w1_d, b1_d, w2_d, b2_d)
    return out2.reshape(e, o)


# speedup vs baseline: 4.0933x; 1.4918x over previous
import functools

import jax
import jax.numpy as jnp
from jax import lax
from jax.experimental import pallas as pl
from jax.experimental.pallas import tpu as pltpu
from jax.experimental.pallas import tpu_sc as plsc

_NC = 2
_NS = 16
_L = 16
_NW = _NC * _NS


def _node_proj_body(feat_ref, wsk_ref, bsk_ref, wrk_ref, brk_ref, vsk_ref, vrk_ref):
    f = feat_ref[...]
    vsk_ref[...] = (
        jnp.dot(f, wsk_ref[...], preferred_element_type=jnp.float32) + bsk_ref[...]
    )
    vrk_ref[...] = (
        jnp.dot(f, wrk_ref[...], preferred_element_type=jnp.float32) + brk_ref[...]
    )


def _edge_mlp_body(vk2_ref, eflo_ref, efhi_ref, wek_ref, bek_ref, w1_ref, b1_ref,
                   w2_ref, b2_ref, out_ref):
    o = out_ref.shape[2]
    efc = jnp.concatenate([eflo_ref[...], efhi_ref[...]], axis=1)
    ekc = jnp.dot(efc, wek_ref[...], preferred_element_type=jnp.float32)
    a = jnp.maximum(vk2_ref[...] + ekc + bek_ref[...], 0.0)
    a = jnp.maximum(
        jnp.dot(a, w1_ref[...], preferred_element_type=jnp.float32) + b1_ref[...], 0.0
    )
    a = jnp.maximum(
        jnp.dot(a, w2_ref[...], preferred_element_type=jnp.float32) + b2_ref[...], 0.0
    )
    out_ref[0] = a[:, :o]
    out_ref[1] = a[:, o:]


def _make_gather_kernel(n_nodes, n_edges, h, epw, ch):
    nch = epw // ch
    ch2 = 2 * ch
    h2 = 2 * h
    half = n_edges // 2
    rps = (n_nodes // _NS) // 8 * 8
    mesh = plsc.VectorSubcoreMesh(
        core_axis_name="c", subcore_axis_name="s", num_cores=_NC, num_subcores=_NS
    )

    @functools.partial(
        pl.kernel,
        out_type=jax.ShapeDtypeStruct((n_edges // 2, h2), jnp.float32),
        mesh=mesh,
        scratch_types=[
            pltpu.VMEM_SHARED((n_nodes, h), jnp.float32),
            pltpu.VMEM_SHARED((n_nodes, h), jnp.float32),
            pltpu.VMEM((2 * epw,), jnp.int32),
            pltpu.VMEM((2 * epw,), jnp.int32),
            pltpu.VMEM((ch2, h), jnp.float32),
            pltpu.VMEM((ch2, h), jnp.float32),
            pltpu.VMEM((ch2, h), jnp.float32),
            pltpu.VMEM((ch2, h), jnp.float32),
            pltpu.VMEM((ch, h2), jnp.float32),
            pltpu.VMEM((ch, h2), jnp.float32),
            pltpu.SemaphoreType.DMA,
            pltpu.SemaphoreType.DMA,
            pltpu.SemaphoreType.DMA,
        ],
        compiler_params=pltpu.CompilerParams(use_tc_tiling_on_sc=False),
    )
    def gather_add(vsk_hbm, vrk_hbm, ei_hbm, out_hbm,
                   vsk_sh, vrk_sh, sidx, didx, rs0, rd0, rs1, rd1, ov0, ov1,
                   semg0, semg1, semo):
        cid = lax.axis_index("c")
        sid = lax.axis_index("s")
        wid = sid * _NC + cid
        base = wid * epw

        for s in range(_NS):
            sz = rps if s < _NS - 1 else n_nodes - rps * (_NS - 1)

            @pl.when(sid == s)
            def _stage(s=s, sz=sz):
                pltpu.sync_copy(vsk_hbm.at[pl.ds(s * rps, sz)],
                                vsk_sh.at[pl.ds(s * rps, sz)])
                pltpu.sync_copy(vrk_hbm.at[pl.ds(s * rps, sz)],
                                vrk_sh.at[pl.ds(s * rps, sz)])

        plsc.subcore_barrier()

        pltpu.sync_copy(ei_hbm.at[0, pl.ds(base, epw)], sidx.at[pl.ds(0, epw)])
        pltpu.sync_copy(ei_hbm.at[0, pl.ds(half + base, epw)],
                        sidx.at[pl.ds(epw, epw)])
        pltpu.sync_copy(ei_hbm.at[1, pl.ds(base, epw)], didx.at[pl.ds(0, epw)])
        pltpu.sync_copy(ei_hbm.at[1, pl.ds(half + base, epw)],
                        didx.at[pl.ds(epw, epw)])

        def issue(j, rs, rd, semg):
            io = pl.multiple_of(j * ch, 8)
            ih = pl.multiple_of(epw + j * ch, 8)
            c0 = pltpu.async_copy(vsk_sh.at[sidx.at[pl.ds(io, ch)]],
                                  rs.at[pl.ds(0, ch)], semg)
            c1 = pltpu.async_copy(vsk_sh.at[sidx.at[pl.ds(ih, ch)]],
                                  rs.at[pl.ds(ch, ch)], semg)
            c2 = pltpu.async_copy(vrk_sh.at[didx.at[pl.ds(io, ch)]],
                                  rd.at[pl.ds(0, ch)], semg)
            c3 = pltpu.async_copy(vrk_sh.at[didx.at[pl.ds(ih, ch)]],
                                  rd.at[pl.ds(ch, ch)], semg)
            return c0, c1, c2, c3

        def combine(rs, rd, ov):
            @plsc.parallel_loop(0, ch, unroll=4)
            def add_rows(r):
                for c in range(h // _L):
                    sl = pl.ds(c * _L, _L)
                    sr = pl.ds(h + c * _L, _L)
                    ov[r, sl] = rs[r, sl] + rd[r, sl]
                    ov[r, sr] = rs[ch + r, sl] + rd[ch + r, sl]

        def flush(j, ov):
            oo = pl.multiple_of(base + j * ch, 8)
            return pltpu.async_copy(ov, out_hbm.at[pl.ds(oo, ch)], semo)

        def pair_body(i, carry):
            j0 = 2 * i
            j1 = 2 * i + 1
            g0 = issue(j0, rs0, rd0, semg0)
            g1 = issue(j1, rs1, rd1, semg1)
            for c in g0:
                c.wait()
            combine(rs0, rd0, ov0)
            o0 = flush(j0, ov0)
            for c in g1:
                c.wait()
            combine(rs1, rd1, ov1)
            o1 = flush(j1, ov1)
            o0.wait()
            o1.wait()
            return carry

        lax.fori_loop(0, nch // 2, pair_body, 0)

        if nch % 2:
            j = nch - 1
            g = issue(j, rs0, rd0, semg0)
            for c in g:
                c.wait()
            combine(rs0, rd0, ov0)
            flush(j, ov0).wait()

    return gather_add


def _blkdiag(w):
    r, c = w.shape
    z = jnp.zeros((2 * r, 2 * c), w.dtype)
    return z.at[:r, :c].set(w).at[r:, c:].set(w)


def kernel(feat, efeat, edge_index, W_vsk, b_vsk, W_vrk, b_vrk, W_ek, b_ek, W1, b1,
           W2, b2):
    n, f_in = feat.shape
    e = efeat.shape[0]
    h = W_vsk.shape[0]
    o = W2.shape[0]
    half = e // 2

    vsk, vrk = pl.pallas_call(
        _node_proj_body,
        out_shape=[
            jax.ShapeDtypeStruct((n, h), jnp.float32),
            jax.ShapeDtypeStruct((n, h), jnp.float32),
        ],
    )(feat, W_vsk.T, b_vsk[None, :], W_vrk.T, b_vrk[None, :])

    epw = half // _NW
    ch = 40
    gather_add = _make_gather_kernel(n, e, h, epw, ch)
    vk2 = gather_add(vsk, vrk, edge_index)

    beh = 1600
    nblk = half // beh
    wek_d = _blkdiag(W_ek.T)
    bek_d = jnp.concatenate([b_ek, b_ek])[None, :]
    w1_d = _blkdiag(W1.T)
    b1_d = jnp.concatenate([b1, b1])[None, :]
    w2_d = _blkdiag(W2.T)
    b2_d = jnp.concatenate([b2, b2])[None, :]
    out2 = pl.pallas_call(
        _edge_mlp_body,
        grid=(nblk,),
        in_specs=[
            pl.BlockSpec((beh, 2 * h), lambda i: (i, 0)),
            pl.BlockSpec((beh, f_in), lambda i: (i, 0)),
            pl.BlockSpec((beh, f_in), lambda i, _n=nblk: (i + _n, 0)),
            pl.BlockSpec((2 * f_in, 2 * h), lambda i: (0, 0)),
            pl.BlockSpec((1, 2 * h), lambda i: (0, 0)),
            pl.BlockSpec((2 * h, 2 * h), lambda i: (0, 0)),
            pl.BlockSpec((1, 2 * h), lambda i: (0, 0)),
            pl.BlockSpec((2 * h, 2 * o), lambda i: (0, 0)),
            pl.BlockSpec((1, 2 * o), lambda i: (0, 0)),
        ],
        out_specs=pl.BlockSpec((2, beh, o), lambda i: (0, i, 0)),
        out_shape=jax.ShapeDtypeStruct((2, half, o), jnp.float32),
        compiler_params=pltpu.CompilerParams(
            dimension_semantics=("arbitrary",),
        ),
    )(vk2, efeat, efeat, wek_d, bek_d, w1_d, b1_d, w2_d, b2_d)
    return out2.reshape(e, o)


# speedup vs baseline: 4.1735x; 1.0196x over previous
import functools

import jax
import jax.numpy as jnp
from jax import lax
from jax.experimental import pallas as pl
from jax.experimental.pallas import tpu as pltpu
from jax.experimental.pallas import tpu_sc as plsc

_NC = 2
_NS = 16
_L = 16
_NW = _NC * _NS


def _node_proj_body(feat_ref, w_ref, b_ref, t_ref):
    t_ref[...] = (
        jnp.dot(feat_ref[...], w_ref[...], preferred_element_type=jnp.float32)
        + b_ref[...]
    )


def _edge_mlp_body(vk2_ref, eflo_ref, efhi_ref, wek_ref, bek_ref, w1_ref, b1_ref,
                   w2_ref, b2_ref, out_ref):
    o = out_ref.shape[2]
    efc = jnp.concatenate([eflo_ref[...], efhi_ref[...]], axis=1)
    ekc = jnp.dot(efc, wek_ref[...], preferred_element_type=jnp.float32)
    a = jnp.maximum(vk2_ref[...] + ekc + bek_ref[...], 0.0)
    a = jnp.maximum(
        jnp.dot(a, w1_ref[...], preferred_element_type=jnp.float32) + b1_ref[...], 0.0
    )
    a = jnp.maximum(
        jnp.dot(a, w2_ref[...], preferred_element_type=jnp.float32) + b2_ref[...], 0.0
    )
    out_ref[0] = a[:, :o]
    out_ref[1] = a[:, o:]


def _make_gather_kernel(n_nodes, n_edges, h, epw, ch):
    nch = epw // ch
    ch2 = 2 * ch
    h2 = 2 * h
    half = n_edges // 2
    rps = (n_nodes // _NS) // 8 * 8
    mesh = plsc.VectorSubcoreMesh(
        core_axis_name="c", subcore_axis_name="s", num_cores=_NC, num_subcores=_NS
    )

    @functools.partial(
        pl.kernel,
        out_type=jax.ShapeDtypeStruct((n_edges // 2, h2), jnp.float32),
        mesh=mesh,
        scratch_types=[
            pltpu.VMEM_SHARED((n_nodes, h), jnp.float32),
            pltpu.VMEM_SHARED((n_nodes, h), jnp.float32),
            pltpu.VMEM((2 * epw,), jnp.int32),
            pltpu.VMEM((2 * epw,), jnp.int32),
            pltpu.VMEM((ch2, h), jnp.float32),
            pltpu.VMEM((ch2, h), jnp.float32),
            pltpu.VMEM((ch2, h), jnp.float32),
            pltpu.VMEM((ch2, h), jnp.float32),
            pltpu.VMEM((ch, h2), jnp.float32),
            pltpu.VMEM((ch, h2), jnp.float32),
            pltpu.SemaphoreType.DMA,
            pltpu.SemaphoreType.DMA,
            pltpu.SemaphoreType.DMA,
        ],
        compiler_params=pltpu.CompilerParams(use_tc_tiling_on_sc=False),
    )
    def gather_add(t_hbm, ei_hbm, out_hbm,
                   vsk_sh, vrk_sh, sidx, didx, rs0, rd0, rs1, rd1, ov0, ov1,
                   semg0, semg1, semo):
        cid = lax.axis_index("c")
        sid = lax.axis_index("s")
        wid = sid * _NC + cid
        base = wid * epw

        for s in range(_NS):
            sz = rps if s < _NS - 1 else n_nodes - rps * (_NS - 1)

            @pl.when(sid == s)
            def _stage(s=s, sz=sz):
                pltpu.sync_copy(t_hbm.at[pl.ds(s * rps, sz), pl.ds(0, h)],
                                vsk_sh.at[pl.ds(s * rps, sz)])
                pltpu.sync_copy(t_hbm.at[pl.ds(s * rps, sz), pl.ds(h, h)],
                                vrk_sh.at[pl.ds(s * rps, sz)])

        plsc.subcore_barrier()

        pltpu.sync_copy(ei_hbm.at[0, pl.ds(base, epw)], sidx.at[pl.ds(0, epw)])
        pltpu.sync_copy(ei_hbm.at[0, pl.ds(half + base, epw)],
                        sidx.at[pl.ds(epw, epw)])
        pltpu.sync_copy(ei_hbm.at[1, pl.ds(base, epw)], didx.at[pl.ds(0, epw)])
        pltpu.sync_copy(ei_hbm.at[1, pl.ds(half + base, epw)],
                        didx.at[pl.ds(epw, epw)])

        def issue(j, rs, rd, semg):
            io = pl.multiple_of(j * ch, 8)
            ih = pl.multiple_of(epw + j * ch, 8)
            c0 = pltpu.async_copy(vsk_sh.at[sidx.at[pl.ds(io, ch)]],
                                  rs.at[pl.ds(0, ch)], semg)
            c1 = pltpu.async_copy(vsk_sh.at[sidx.at[pl.ds(ih, ch)]],
                                  rs.at[pl.ds(ch, ch)], semg)
            c2 = pltpu.async_copy(vrk_sh.at[didx.at[pl.ds(io, ch)]],
                                  rd.at[pl.ds(0, ch)], semg)
            c3 = pltpu.async_copy(vrk_sh.at[didx.at[pl.ds(ih, ch)]],
                                  rd.at[pl.ds(ch, ch)], semg)
            return c0, c1, c2, c3

        def combine(rs, rd, ov):
            @plsc.parallel_loop(0, ch, unroll=4)
            def add_rows(r):
                for c in range(h // _L):
                    sl = pl.ds(c * _L, _L)
                    sr = pl.ds(h + c * _L, _L)
                    ov[r, sl] = rs[r, sl] + rd[r, sl]
                    ov[r, sr] = rs[ch + r, sl] + rd[ch + r, sl]

        def flush(j, ov):
            oo = pl.multiple_of(base + j * ch, 8)
            return pltpu.async_copy(ov, out_hbm.at[pl.ds(oo, ch)], semo)

        def pair_body(i, carry):
            j0 = 2 * i
            j1 = 2 * i + 1
            g0 = issue(j0, rs0, rd0, semg0)
            g1 = issue(j1, rs1, rd1, semg1)
            for c in g0:
                c.wait()
            combine(rs0, rd0, ov0)
            o0 = flush(j0, ov0)
            for c in g1:
                c.wait()
            combine(rs1, rd1, ov1)
            o1 = flush(j1, ov1)
            o0.wait()
            o1.wait()
            return carry

        lax.fori_loop(0, nch // 2, pair_body, 0)

        if nch % 2:
            j = nch - 1
            g = issue(j, rs0, rd0, semg0)
            for c in g:
                c.wait()
            combine(rs0, rd0, ov0)
            flush(j, ov0).wait()

    return gather_add


def _blkdiag(w):
    r, c = w.shape
    z = jnp.zeros((2 * r, 2 * c), w.dtype)
    return z.at[:r, :c].set(w).at[r:, c:].set(w)


def kernel(feat, efeat, edge_index, W_vsk, b_vsk, W_vrk, b_vrk, W_ek, b_ek, W1, b1,
           W2, b2):
    n, f_in = feat.shape
    e = efeat.shape[0]
    h = W_vsk.shape[0]
    o = W2.shape[0]
    half = e // 2

    w_cat = jnp.concatenate([W_vsk.T, W_vrk.T], axis=1)
    b_cat = jnp.concatenate([b_vsk, b_vrk])[None, :]
    t_tab = pl.pallas_call(
        _node_proj_body,
        out_shape=jax.ShapeDtypeStruct((n, 2 * h), jnp.float32),
    )(feat, w_cat, b_cat)

    epw = half // _NW
    ch = 40
    gather_add = _make_gather_kernel(n, e, h, epw, ch)
    vk2 = gather_add(t_tab, edge_index)

    beh = 1600
    nblk = half // beh
    wek_d = _blkdiag(W_ek.T)
    bek_d = jnp.concatenate([b_ek, b_ek])[None, :]
    w1_d = _blkdiag(W1.T)
    b1_d = jnp.concatenate([b1, b1])[None, :]
    w2_d = _blkdiag(W2.T)
    b2_d = jnp.concatenate([b2, b2])[None, :]
    out2 = pl.pallas_call(
        _edge_mlp_body,
        grid=(nblk,),
        in_specs=[
            pl.BlockSpec((beh, 2 * h), lambda i: (i, 0)),
            pl.BlockSpec((beh, f_in), lambda i: (i, 0)),
            pl.BlockSpec((beh, f_in), lambda i, _n=nblk: (i + _n, 0)),
            pl.BlockSpec((2 * f_in, 2 * h), lambda i: (0, 0)),
            pl.BlockSpec((1, 2 * h), lambda i: (0, 0)),
            pl.BlockSpec((2 * h, 2 * h), lambda i: (0, 0)),
            pl.BlockSpec((1, 2 * h), lambda i: (0, 0)),
            pl.BlockSpec((2 * h, 2 * o), lambda i: (0, 0)),
            pl.BlockSpec((1, 2 * o), lambda i: (0, 0)),
        ],
        out_specs=pl.BlockSpec((2, beh, o), lambda i: (0, i, 0)),
        out_shape=jax.ShapeDtypeStruct((2, half, o), jnp.float32),
        compiler_params=pltpu.CompilerParams(
            dimension_semantics=("arbitrary",),
        ),
    )(vk2, efeat, efeat, wek_d, bek_d, w1_d, b1_d, w2_d, b2_d)
    return out2.reshape(e, o)
